# baseline (speedup 1.0000x reference)
import functools

import jax
import jax.numpy as jnp
from jax.experimental import pallas as pl
from jax.experimental.pallas import tpu as pltpu

_EPS = 1e-05


def _fused_block(x_ref, ps_ref, pb_ref, beta_ref, w_ref, o_ref, *, tc, cgn):
    xb = x_ref[0]
    si = jnp.concatenate([xb, xb * xb], axis=1)
    s = jnp.dot(ps_ref[...], si, preferred_element_type=jnp.float32)
    mean = s[:, :tc]
    ex2 = s[:, tc:]
    var = (ex2 - mean * mean) * (cgn / (cgn - 1.0))
    inv = jax.lax.rsqrt(var + _EPS)
    cc = jnp.concatenate([inv, -mean * inv], axis=1)
    ab = jnp.dot(pb_ref[...], cc, preferred_element_type=jnp.float32)
    beta = pltpu.repeat(beta_ref[...], tc // 128, axis=1)
    h = jnp.maximum(xb * ab[:, :tc] + ab[:, tc:] + beta, 0.0)
    hb = h.astype(jnp.bfloat16)
    o_ref[0] = xb + jnp.dot(w_ref[...], hb, preferred_element_type=jnp.float32)


def kernel(x, gamma, beta, w_fc0):
    b, d, t = x.shape
    groups = 8
    cg = d // groups
    gn = groups * 4
    cgn = d // gn

    wg = w_fc0.reshape(groups, cg, cg)
    w_bd = (wg[:, :, None, :] * jnp.eye(groups, dtype=w_fc0.dtype)[:, None, :, None])
    w_bd = w_bd.reshape(d, d).astype(jnp.bfloat16)

    eye_gn = jnp.eye(gn, dtype=x.dtype)
    pstat = jnp.repeat(eye_gn, cgn, axis=1) * (1.0 / cgn)
    pb = jnp.repeat(eye_gn, cgn, axis=0) * gamma.reshape(d, 1)
    b2 = jnp.broadcast_to(beta.reshape(d, 1), (d, 128))

    tc = min(512, t)
    grid = (b, t // tc)
    body = functools.partial(_fused_block, tc=tc, cgn=float(cgn))

    return pl.pallas_call(
        body,
        grid=grid,
        in_specs=[
            pl.BlockSpec((1, d, tc), lambda i, j: (i, 0, j)),
            pl.BlockSpec((gn, d), lambda i, j: (0, 0)),
            pl.BlockSpec((d, gn), lambda i, j: (0, 0)),
            pl.BlockSpec((d, 128), lambda i, j: (0, 0)),
            pl.BlockSpec((d, d), lambda i, j: (0, 0)),
        ],
        out_specs=pl.BlockSpec((1, d, tc), lambda i, j: (i, 0, j)),
        out_shape=jax.ShapeDtypeStruct((b, d, t), x.dtype),
        compiler_params=pltpu.CompilerParams(
            dimension_semantics=("parallel", "parallel"),
        ),
    )(x, pstat, pb, b2, w_bd)


# speedup vs baseline: 2.0622x; 2.0622x over previous
import functools

import jax
import jax.numpy as jnp
from jax.experimental import pallas as pl
from jax.experimental.pallas import tpu as pltpu

_EPS = 1e-05


def _fused_block(x_ref, ps_ref, pb_ref, beta_ref, w_ref, o_ref, *, tc, cgn):
    xb = x_ref[0]
    mean = jnp.dot(ps_ref[...], xb, preferred_element_type=jnp.float32)
    ex2 = jnp.dot(ps_ref[...], xb * xb, preferred_element_type=jnp.float32)
    var = (ex2 - mean * mean) * (cgn / (cgn - 1.0))
    inv = jax.lax.rsqrt(var + _EPS)
    a = jnp.dot(pb_ref[...], inv, preferred_element_type=jnp.float32)
    c = jnp.dot(pb_ref[...], -mean * inv, preferred_element_type=jnp.float32)
    beta = pltpu.repeat(beta_ref[...], tc // 128, axis=1)
    h = jnp.maximum(xb * a + c + beta, 0.0)
    hb = h.astype(jnp.bfloat16)
    o_ref[0] = xb + jnp.dot(w_ref[...], hb, preferred_element_type=jnp.float32)


def kernel(x, gamma, beta, w_fc0):
    b, d, t = x.shape
    groups = 8
    cg = d // groups
    gn = groups * 4
    cgn = d // gn

    wg = w_fc0.reshape(groups, cg, cg)
    w_bd = (wg[:, :, None, :] * jnp.eye(groups, dtype=w_fc0.dtype)[:, None, :, None])
    w_bd = w_bd.reshape(d, d).astype(jnp.bfloat16)

    eye_gn = jnp.eye(gn, dtype=x.dtype)
    pstat = jnp.repeat(eye_gn, cgn, axis=1) * (1.0 / cgn)
    pb = jnp.repeat(eye_gn, cgn, axis=0) * gamma.reshape(d, 1)
    b2 = jnp.broadcast_to(beta.reshape(d, 1), (d, 128))

    tc = min(2048, t)
    grid = (b, t // tc)
    body = functools.partial(_fused_block, tc=tc, cgn=float(cgn))

    return pl.pallas_call(
        body,
        grid=grid,
        in_specs=[
            pl.BlockSpec((1, d, tc), lambda i, j: (i, 0, j)),
            pl.BlockSpec((gn, d), lambda i, j: (0, 0)),
            pl.BlockSpec((d, gn), lambda i, j: (0, 0)),
            pl.BlockSpec((d, 128), lambda i, j: (0, 0)),
            pl.BlockSpec((d, d), lambda i, j: (0, 0)),
        ],
        out_specs=pl.BlockSpec((1, d, tc), lambda i, j: (i, 0, j)),
        out_shape=jax.ShapeDtypeStruct((b, d, t), x.dtype),
        compiler_params=pltpu.CompilerParams(
            dimension_semantics=("parallel", "parallel"),
        ),
    )(x, pstat, pb, b2, w_bd)


# speedup vs baseline: 2.4068x; 1.1671x over previous
import functools

import jax
import jax.numpy as jnp
from jax.experimental import pallas as pl
from jax.experimental.pallas import tpu as pltpu

_EPS = 1e-05


def _fused_block(x_ref, ps_ref, pb_ref, beta_ref, w_ref, o_ref, *, tc, cgn):
    xb = x_ref[0]
    mean = jnp.dot(ps_ref[...], xb, preferred_element_type=jnp.float32)
    ex2 = jnp.dot(ps_ref[...], xb * xb, preferred_element_type=jnp.float32)
    var = (ex2 - mean * mean) * (cgn / (cgn - 1.0))
    inv = jax.lax.rsqrt(var + _EPS)
    a = jnp.dot(pb_ref[...], inv, preferred_element_type=jnp.float32)
    c = jnp.dot(pb_ref[...], -mean * inv, preferred_element_type=jnp.float32)
    beta = pltpu.repeat(beta_ref[...], tc // 128, axis=1)
    h = jnp.maximum(xb * a + c + beta, 0.0)
    hb = h.astype(jnp.bfloat16)
    o_ref[0] = xb + jnp.dot(w_ref[...], hb, preferred_element_type=jnp.float32)


def kernel(x, gamma, beta, w_fc0):
    b, d, t = x.shape
    groups = 8
    cg = d // groups
    gn = groups * 4
    cgn = d // gn

    wg = w_fc0.reshape(groups, cg, cg)
    w_bd = (wg[:, :, None, :] * jnp.eye(groups, dtype=w_fc0.dtype)[:, None, :, None])
    w_bd = w_bd.reshape(d, d).astype(jnp.bfloat16)

    eye_gn = jnp.eye(gn, dtype=x.dtype)
    pstat = jnp.repeat(eye_gn, cgn, axis=1) * (1.0 / cgn)
    pb = jnp.repeat(eye_gn, cgn, axis=0) * gamma.reshape(d, 1)
    b2 = jnp.broadcast_to(beta.reshape(d, 1), (d, 128))

    tc = min(4096, t)
    grid = (b, t // tc)
    body = functools.partial(_fused_block, tc=tc, cgn=float(cgn))

    return pl.pallas_call(
        body,
        grid=grid,
        in_specs=[
            pl.BlockSpec((1, d, tc), lambda i, j: (i, 0, j)),
            pl.BlockSpec((gn, d), lambda i, j: (0, 0)),
            pl.BlockSpec((d, gn), lambda i, j: (0, 0)),
            pl.BlockSpec((d, 128), lambda i, j: (0, 0)),
            pl.BlockSpec((d, d), lambda i, j: (0, 0)),
        ],
        out_specs=pl.BlockSpec((1, d, tc), lambda i, j: (i, 0, j)),
        out_shape=jax.ShapeDtypeStruct((b, d, t), x.dtype),
        compiler_params=pltpu.CompilerParams(
            dimension_semantics=("parallel", "parallel"),
        ),
    )(x, pstat, pb, b2, w_bd)


# speedup vs baseline: 2.6387x; 1.0964x over previous
import functools

import jax
import jax.numpy as jnp
from jax.experimental import pallas as pl
from jax.experimental.pallas import tpu as pltpu

_EPS = 1e-05


def _fused_block(x_ref, ps_ref, pb_ref, beta_ref, w_ref, o_ref, *, tc, cgn):
    xb = x_ref[0]
    mean = jnp.dot(ps_ref[...], xb, preferred_element_type=jnp.float32)
    ex2 = jnp.dot(ps_ref[...], xb * xb, preferred_element_type=jnp.float32)
    var = (ex2 - mean * mean) * (cgn / (cgn - 1.0))
    inv = jax.lax.rsqrt(var + _EPS)
    a = jnp.dot(pb_ref[...], inv, preferred_element_type=jnp.float32)
    c = jnp.dot(pb_ref[...], -mean * inv, preferred_element_type=jnp.float32)
    beta = pltpu.repeat(beta_ref[...], tc // 128, axis=1)
    h = jnp.maximum(xb * a + c + beta, 0.0)
    hb = h.astype(jnp.bfloat16)
    o_ref[0] = xb + jnp.dot(w_ref[...], hb, preferred_element_type=jnp.float32)


def kernel(x, gamma, beta, w_fc0):
    b, d, t = x.shape
    groups = 8
    cg = d // groups
    gn = groups * 4
    cgn = d // gn

    wg = w_fc0.reshape(groups, cg, cg)
    w_bd = (wg[:, :, None, :] * jnp.eye(groups, dtype=w_fc0.dtype)[:, None, :, None])
    w_bd = w_bd.reshape(d, d).astype(jnp.bfloat16)

    eye_gn = jnp.eye(gn, dtype=x.dtype)
    pstat = jnp.repeat(eye_gn, cgn, axis=1) * (1.0 / cgn)
    pb = jnp.repeat(eye_gn, cgn, axis=0) * gamma.reshape(d, 1)
    b2 = jnp.broadcast_to(beta.reshape(d, 1), (d, 128))

    tc = min(8192, t)
    grid = (b, t // tc)
    body = functools.partial(_fused_block, tc=tc, cgn=float(cgn))

    return pl.pallas_call(
        body,
        grid=grid,
        in_specs=[
            pl.BlockSpec((1, d, tc), lambda i, j: (i, 0, j)),
            pl.BlockSpec((gn, d), lambda i, j: (0, 0)),
            pl.BlockSpec((d, gn), lambda i, j: (0, 0)),
            pl.BlockSpec((d, 128), lambda i, j: (0, 0)),
            pl.BlockSpec((d, d), lambda i, j: (0, 0)),
        ],
        out_specs=pl.BlockSpec((1, d, tc), lambda i, j: (i, 0, j)),
        out_shape=jax.ShapeDtypeStruct((b, d, t), x.dtype),
        compiler_params=pltpu.CompilerParams(
            dimension_semantics=("parallel", "parallel"),
        ),
    )(x, pstat, pb, b2, w_bd)
